# baseline (speedup 1.0000x reference)
import functools

import jax
import jax.numpy as jnp
from jax import lax
from jax.experimental import pallas as pl
from jax.experimental.pallas import tpu as pltpu
from jax.experimental.pallas import tpu_sc as plsc

_B, _T, _D, _K = 16, 1024, 256, 8192
_N = _B * _T
_BM = 1024
_NB = _N // _BM
_TB = _T // _BM


def _prep_body(cw_ref, cn_ref, cwb_ref, cwt_ref):
    cw = cw_ref[...]
    cn_ref[...] = jnp.sum(cw ** 2, axis=0, keepdims=True)
    cwb_ref[...] = cw.astype(jnp.bfloat16)
    cwt_ref[...] = cw.T


_prep_call = pl.pallas_call(
    _prep_body,
    out_shape=[
        jax.ShapeDtypeStruct((1, _K), jnp.float32),
        jax.ShapeDtypeStruct((_D, _K), jnp.bfloat16),
        jax.ShapeDtypeStruct((_K, _D), jnp.float32),
    ],
)


def _dist_argmin_body(len_ref, x_ref, cwb_ref, cn_ref, idx_ref, diff_ref,
                      acc_ref):
    rb = pl.program_id(0)
    x = x_ref[...]
    xm2 = (-2.0 * x).astype(jnp.bfloat16)
    x_norm = jnp.sum(x ** 2, axis=1, keepdims=True)

    rt = lambda t: t.astype(jnp.bfloat16).astype(jnp.float32)
    acc_v = acc_x = acc_i = None
    for lo in (0, 2816, 5632):
        hi = min(lo + 2816, _K)
        ndot = jnp.dot(xm2, cwb_ref[:, lo:hi],
                       preferred_element_type=jnp.float32)
        seg = (x_norm + ndot) + cn_ref[:, lo:hi]
        v = jnp.min(seg, axis=1, keepdims=True)
        i = jnp.argmin(seg, axis=1) + lo
        if acc_v is None:
            acc_v, acc_x, acc_i = v, v, i
        else:
            acc_c = rt(acc_v)
            take = v < acc_c
            acc_i = jnp.where(take[:, 0], i, acc_i)
            acc_x = jnp.where(take, v, acc_x)
            acc_v = jnp.where(take, rt(v), acc_c)
    mind = acc_x
    idx_ref[...] = acc_i.reshape(1, 1, _BM)

    b = rb // _TB
    t0 = (rb % _TB) * _BM
    lb = len_ref[b]
    tpos = t0 + lax.broadcasted_iota(jnp.int32, (_BM, 1), 0)
    contrib = jnp.sum(jnp.where(tpos < lb, mind, 0.0))

    @pl.when(rb == 0)
    def _init():
        acc_ref[0] = 0.0

    acc_ref[0] += contrib

    @pl.when(rb == _NB - 1)
    def _fin():
        total = len_ref[0]
        for i in range(1, _B):
            total += len_ref[i]
        denom = jnp.maximum(total.astype(jnp.float32) * float(_D), 1.0)
        diff_ref[0] = acc_ref[0] / denom


_tc_call = pl.pallas_call(
    _dist_argmin_body,
    grid=(_NB,),
    in_specs=[
        pl.BlockSpec(memory_space=pltpu.SMEM),
        pl.BlockSpec((_BM, _D), lambda rb: (rb, 0)),
        pl.BlockSpec((_D, _K), lambda rb: (0, 0)),
        pl.BlockSpec((1, _K), lambda rb: (0, 0)),
    ],
    out_specs=[
        pl.BlockSpec((1, 1, _BM), lambda rb: (rb, 0, 0)),
        pl.BlockSpec(memory_space=pltpu.SMEM),
    ],
    out_shape=[
        jax.ShapeDtypeStruct((_NB, 1, _BM), jnp.int32),
        jax.ShapeDtypeStruct((1,), jnp.float32),
    ],
    scratch_shapes=[pltpu.SMEM((1,), jnp.float32)],
    compiler_params=pltpu.CompilerParams(dimension_semantics=("arbitrary",)),
)


_NC, _NS = 2, 16
_NW = _NC * _NS
_BPW = _N // _NW
_CH = 128
_NCHUNK = _BPW // _CH


@functools.lru_cache(maxsize=None)
def _get_sc_gather():
    mesh = plsc.VectorSubcoreMesh(core_axis_name="c", subcore_axis_name="s")

    @functools.partial(
        pl.kernel,
        mesh=mesh,
        out_type=jax.ShapeDtypeStruct((_N, _D), jnp.float32),
        scratch_types=[
            pltpu.VMEM((_BPW,), jnp.int32),
            pltpu.VMEM((3, _CH, _D), jnp.float32),
            pltpu.SemaphoreType.DMA,
            pltpu.SemaphoreType.DMA,
            pltpu.SemaphoreType.DMA,
        ],
    )
    def _sc_gather(table_hbm, idx_hbm, out_hbm, idx_v, rows_v, s0, s1, s2):
        wid = lax.axis_index("s") * _NC + lax.axis_index("c")
        base = wid * _BPW
        sems = (s0, s1, s2)
        pltpu.sync_copy(idx_hbm.at[pl.ds(base, _BPW)], idx_v)

        def start(c):
            return pltpu.async_copy(
                table_hbm.at[idx_v.at[pl.ds(c * _CH, _CH)]],
                rows_v.at[c % 3], sems[c % 3])

        copies = [start(c) for c in range(min(3, _NCHUNK))]
        for c in range(_NCHUNK):
            copies[c].wait()
            pltpu.sync_copy(rows_v.at[c % 3],
                            out_hbm.at[pl.ds(base + c * _CH, _CH)])
            if c + 3 < _NCHUNK:
                copies.append(start(c + 3))

    return _sc_gather


def kernel(embedding, length, codewords):
    x = embedding.reshape(_N, _D)
    cn, cwb, cwt = _prep_call(codewords)
    idx3, diff1 = _tc_call(length, x, cwb, cn)
    idx_flat = idx3.reshape(_N)
    quantize = _get_sc_gather()(cwt, idx_flat)
    return (quantize.reshape(_B, _T, _D), diff1[0],
            idx_flat.reshape(_B, _T))


# speedup vs baseline: 1.0459x; 1.0459x over previous
import functools

import jax
import jax.numpy as jnp
from jax import lax
from jax.experimental import pallas as pl
from jax.experimental.pallas import tpu as pltpu
from jax.experimental.pallas import tpu_sc as plsc

_B, _T, _D, _K = 16, 1024, 256, 8192
_N = _B * _T
_BM = 512
_NB = _N // _BM
_TB = _T // _BM


def _prep_body(cw_ref, cn_ref, cwb_ref, cwt_ref):
    cw = cw_ref[...]
    cn_ref[...] = jnp.sum(cw ** 2, axis=0, keepdims=True)
    cwb_ref[...] = cw.astype(jnp.bfloat16)
    cwt_ref[...] = cw.T


_prep_call = pl.pallas_call(
    _prep_body,
    out_shape=[
        jax.ShapeDtypeStruct((1, _K), jnp.float32),
        jax.ShapeDtypeStruct((_D, _K), jnp.bfloat16),
        jax.ShapeDtypeStruct((_K, _D), jnp.float32),
    ],
)


def _dist_argmin_body(len_ref, x_ref, cwb_ref, cn_ref, idx_ref, diff_ref,
                      acc_ref):
    rb = pl.program_id(0)
    x = x_ref[...]
    xm2 = (-2.0 * x).astype(jnp.bfloat16)
    x_norm = jnp.sum(x ** 2, axis=1, keepdims=True)

    rt = lambda t: t.astype(jnp.bfloat16).astype(jnp.float32)
    acc_v = acc_x = acc_i = None
    for lo in (0, 2816, 5632):
        hi = min(lo + 2816, _K)
        ndot = jnp.dot(xm2, cwb_ref[:, lo:hi],
                       preferred_element_type=jnp.float32)
        seg = (x_norm + ndot) + cn_ref[:, lo:hi]
        v = jnp.min(seg, axis=1, keepdims=True)
        i = jnp.argmin(seg, axis=1) + lo
        if acc_v is None:
            acc_v, acc_x, acc_i = v, v, i
        else:
            acc_c = rt(acc_v)
            take = v < acc_c
            acc_i = jnp.where(take[:, 0], i, acc_i)
            acc_x = jnp.where(take, v, acc_x)
            acc_v = jnp.where(take, rt(v), acc_c)
    mind = acc_x
    idx_ref[...] = acc_i.reshape(1, 1, _BM)

    b = rb // _TB
    t0 = (rb % _TB) * _BM
    lb = len_ref[b]
    tpos = t0 + lax.broadcasted_iota(jnp.int32, (_BM, 1), 0)
    contrib = jnp.sum(jnp.where(tpos < lb, mind, 0.0))

    @pl.when(rb == 0)
    def _init():
        acc_ref[0] = 0.0

    acc_ref[0] += contrib

    @pl.when(rb == _NB - 1)
    def _fin():
        total = len_ref[0]
        for i in range(1, _B):
            total += len_ref[i]
        denom = jnp.maximum(total.astype(jnp.float32) * float(_D), 1.0)
        diff_ref[0] = acc_ref[0] / denom


_tc_call = pl.pallas_call(
    _dist_argmin_body,
    grid=(_NB,),
    in_specs=[
        pl.BlockSpec(memory_space=pltpu.SMEM),
        pl.BlockSpec((_BM, _D), lambda rb: (rb, 0)),
        pl.BlockSpec((_D, _K), lambda rb: (0, 0)),
        pl.BlockSpec((1, _K), lambda rb: (0, 0)),
    ],
    out_specs=[
        pl.BlockSpec((1, 1, _BM), lambda rb: (rb, 0, 0)),
        pl.BlockSpec(memory_space=pltpu.SMEM),
    ],
    out_shape=[
        jax.ShapeDtypeStruct((_NB, 1, _BM), jnp.int32),
        jax.ShapeDtypeStruct((1,), jnp.float32),
    ],
    scratch_shapes=[pltpu.SMEM((1,), jnp.float32)],
    compiler_params=pltpu.CompilerParams(dimension_semantics=("arbitrary",)),
)


_NC, _NS = 2, 16
_NW = _NC * _NS
_BPW = _N // _NW
_CH = 128
_NCHUNK = _BPW // _CH


@functools.lru_cache(maxsize=None)
def _get_sc_gather():
    mesh = plsc.VectorSubcoreMesh(core_axis_name="c", subcore_axis_name="s")

    @functools.partial(
        pl.kernel,
        mesh=mesh,
        out_type=jax.ShapeDtypeStruct((_N, _D), jnp.float32),
        scratch_types=[
            pltpu.VMEM((_BPW,), jnp.int32),
            pltpu.VMEM((3, _CH, _D), jnp.float32),
            pltpu.SemaphoreType.DMA,
            pltpu.SemaphoreType.DMA,
            pltpu.SemaphoreType.DMA,
        ],
    )
    def _sc_gather(table_hbm, idx_hbm, out_hbm, idx_v, rows_v, s0, s1, s2):
        wid = lax.axis_index("s") * _NC + lax.axis_index("c")
        base = wid * _BPW
        sems = (s0, s1, s2)
        pltpu.sync_copy(idx_hbm.at[pl.ds(base, _BPW)], idx_v)

        def start(c):
            return pltpu.async_copy(
                table_hbm.at[idx_v.at[pl.ds(c * _CH, _CH)]],
                rows_v.at[c % 3], sems[c % 3])

        copies = [start(c) for c in range(min(3, _NCHUNK))]
        for c in range(_NCHUNK):
            copies[c].wait()
            pltpu.sync_copy(rows_v.at[c % 3],
                            out_hbm.at[pl.ds(base + c * _CH, _CH)])
            if c + 3 < _NCHUNK:
                copies.append(start(c + 3))

    return _sc_gather


def kernel(embedding, length, codewords):
    x = embedding.reshape(_N, _D)
    cn, cwb, cwt = _prep_call(codewords)
    idx3, diff1 = _tc_call(length, x, cwb, cn)
    idx_flat = idx3.reshape(_N)
    quantize = _get_sc_gather()(cwt, idx_flat)
    return (quantize.reshape(_B, _T, _D), diff1[0],
            idx_flat.reshape(_B, _T))
